# baseline (speedup 1.0000x reference)
import functools

import jax
import jax.numpy as jnp
from jax import lax
from jax.experimental import pallas as pl
from jax.experimental.pallas import tpu as pltpu
from jax.experimental.pallas import tpu_sc as plsc

_INFO = plsc.get_sparse_core_info()
_NC = _INFO.num_cores
_NS = _INFO.num_subcores
_NW = _NC * _NS


def _chunk_sizes(per_worker, a, b):
  sizes = []
  left = per_worker
  while left > 0:
    want = a if len(sizes) % 2 == 0 else b
    sizes.append(min(want, left))
    left -= sizes[-1]
  return sizes


@functools.lru_cache(maxsize=None)
def _make_gather(batch, seq, D):
  assert (batch * seq) % _NW == 0
  per_worker = batch * seq // _NW
  assert seq % per_worker == 0
  wpr = seq // per_worker
  B = batch * seq
  CA, CB = 64, 56
  sizes = _chunk_sizes(per_worker, CA, CB)
  offs = [sum(sizes[:j]) for j in range(len(sizes))]
  n_chunks = len(sizes)
  assert all(s % 8 == 0 and s <= 128 for s in sizes)
  mesh = plsc.VectorSubcoreMesh(core_axis_name="c", subcore_axis_name="s")

  @functools.partial(
      pl.kernel,
      out_type=jax.ShapeDtypeStruct((batch, seq, D), jnp.float32),
      mesh=mesh,
      scratch_types=[
          pltpu.VMEM((per_worker,), jnp.int32),
          pltpu.VMEM((CA, D), jnp.float32),
          pltpu.VMEM((CB, D), jnp.float32),
      ] + [pltpu.SemaphoreType.DMA] * 4,
  )
  def gather(ids_hbm, table_hbm, out_hbm, idx_v, rows_a, rows_b, g0, g1, s0,
             s1):
    rows = (rows_a, rows_b)
    gsem = (g0, g1)
    ssem = (s0, s1)
    wid = lax.axis_index("s") * _NC + lax.axis_index("c")
    r = wid // wpr
    c = (wid % wpr) * per_worker
    base = wid * per_worker

    head = sizes[0]
    pltpu.sync_copy(ids_hbm.at[pl.ds(base, head)], idx_v.at[pl.ds(0, head)])

    def fire_gather(j):
      b = j % 2
      dst = rows[b] if sizes[j] == rows[b].shape[0] else rows[b].at[
          pl.ds(0, sizes[j])]
      return pltpu.async_copy(
          table_hbm.at[idx_v.at[pl.ds(offs[j], sizes[j])]], dst, gsem[b])

    def fire_store(j):
      b = j % 2
      src = rows[b] if sizes[j] == rows[b].shape[0] else rows[b].at[
          pl.ds(0, sizes[j])]
      return pltpu.async_copy(
          src, out_hbm.at[r, pl.ds(c + offs[j], sizes[j])], ssem[b])

    gd = [None, None]
    sd = [None, None]
    gd[0] = fire_gather(0)
    if n_chunks > 1:
      pltpu.sync_copy(ids_hbm.at[pl.ds(base + head, per_worker - head)],
                      idx_v.at[pl.ds(head, per_worker - head)])
      gd[1] = fire_gather(1)
    for j in range(n_chunks):
      b = j % 2
      gd[b].wait()
      sd[b] = fire_store(j)
      if j + 2 < n_chunks:
        sd[b].wait()
        gd[b] = fire_gather(j + 2)
    for j in range(max(0, n_chunks - 2), n_chunks):
      sd[j % 2].wait()

  return gather


def kernel(position_ids, table):
  batch, seq = position_ids.shape
  ids = position_ids.reshape(-1).astype(jnp.int32)
  return _make_gather(batch, seq, table.shape[1])(ids, table)


# speedup vs baseline: 1.0016x; 1.0016x over previous
import functools

import jax
import jax.numpy as jnp
from jax import lax
from jax.experimental import pallas as pl
from jax.experimental.pallas import tpu as pltpu
from jax.experimental.pallas import tpu_sc as plsc

_INFO = plsc.get_sparse_core_info()
_NC = _INFO.num_cores
_NS = _INFO.num_subcores
_NW = _NC * _NS


def _chunk_sizes(per_worker, a, b):
  sizes = []
  left = per_worker
  while left > 0:
    want = a if len(sizes) % 2 == 0 else b
    sizes.append(min(want, left))
    left -= sizes[-1]
  return sizes


@functools.lru_cache(maxsize=None)
def _make_gather(batch, seq, D):
  assert (batch * seq) % _NW == 0
  per_worker = batch * seq // _NW
  assert seq % per_worker == 0
  wpr = seq // per_worker
  CA, CB = 64, 56
  sizes = _chunk_sizes(per_worker, CA, CB)
  offs = [sum(sizes[:j]) for j in range(len(sizes))]
  n_chunks = len(sizes)
  assert all(s % 8 == 0 and s <= 128 for s in sizes)
  mesh = plsc.VectorSubcoreMesh(core_axis_name="c", subcore_axis_name="s")

  @functools.partial(
      pl.kernel,
      out_type=jax.ShapeDtypeStruct((batch, seq, D), jnp.float32),
      mesh=mesh,
      scratch_types=[
          pltpu.VMEM((per_worker,), jnp.int32),
          pltpu.VMEM((CA, D), jnp.float32),
          pltpu.VMEM((CB, D), jnp.float32),
      ] + [pltpu.SemaphoreType.DMA] * 4,
  )
  def gather(ids_hbm, table_hbm, out_hbm, idx_v, rows_a, rows_b, g0, g1, s0,
             s1):
    rows = (rows_a, rows_b)
    gsem = (g0, g1)
    ssem = (s0, s1)
    wid = lax.axis_index("s") * _NC + lax.axis_index("c")
    r = wid // wpr
    c = (wid % wpr) * per_worker
    base = wid * per_worker

    head = sizes[0]
    pltpu.sync_copy(ids_hbm.at[pl.ds(base, head)], idx_v.at[pl.ds(0, head)])

    def fire_gather(j):
      b = j % 2
      dst = rows[b] if sizes[j] == rows[b].shape[0] else rows[b].at[
          pl.ds(0, sizes[j])]
      return pltpu.async_copy(
          table_hbm.at[idx_v.at[pl.ds(offs[j], sizes[j])]], dst, gsem[b])

    def fire_store(j):
      b = j % 2
      src = rows[b] if sizes[j] == rows[b].shape[0] else rows[b].at[
          pl.ds(0, sizes[j])]
      return pltpu.async_copy(
          src, out_hbm.at[r, pl.ds(c + offs[j], sizes[j])], ssem[b])

    gd = [None, None]
    sd = [None, None]
    gd[0] = fire_gather(0)
    if n_chunks > 1:
      pltpu.sync_copy(ids_hbm.at[pl.ds(base + head, per_worker - head)],
                      idx_v.at[pl.ds(head, per_worker - head)])
      gd[1] = fire_gather(1)
    for j in range(n_chunks):
      b = j % 2
      gd[b].wait()
      sd[b] = fire_store(j)
      if j + 2 < n_chunks:
        sd[b].wait()
        gd[b] = fire_gather(j + 2)
    for j in range(max(0, n_chunks - 2), n_chunks):
      sd[j % 2].wait()

  return gather


def kernel(position_ids, table):
  batch, seq = position_ids.shape
  ids = position_ids.reshape(-1).astype(jnp.int32)
  return _make_gather(batch, seq, table.shape[1])(ids, table)
